# baseline (speedup 1.0000x reference)
import functools
import math

import jax
import jax.numpy as jnp
from jax import lax
from jax.experimental import pallas as pl
from jax.experimental.pallas import tpu as pltpu
from jax.experimental.pallas import tpu_sc as plsc

_N = 10000
_E = 320000
_D = 128
_D_ATTR = 16
_D_EMB = 16
_HID = 8
_AVG = 32.0

_NC = 2
_NS = 16
_NW = _NC * _NS
_B = 128
_EPAD = 327680
_NCHUNK = _EPAD // _B
_NT = _NCHUNK // _NW
_NPAD = 10112
_STRIPE = _NPAD // _NS


def _edge_weight_body(emb_ref, ea_ref, wm0_ref, wm1_ref, out_ref):
    z = jnp.dot(emb_ref[...], wm0_ref[...],
                preferred_element_type=jnp.float32) * (1.0 / math.sqrt(_D_EMB))
    h = z / (1.0 + jnp.exp(-z))
    w = jnp.dot(h, wm1_ref[...], preferred_element_type=jnp.float32)
    out_ref[...] = w * ea_ref[...] * (1.0 / (math.sqrt(_HID) * math.sqrt(_AVG)))


def _edge_weights(emb, ea, wm0, wm1):
    be = 12800
    grid = _E // be
    return pl.pallas_call(
        _edge_weight_body,
        grid=(grid,),
        in_specs=[
            pl.BlockSpec((be, _D_EMB), lambda i: (i, 0)),
            pl.BlockSpec((be, 1), lambda i: (i, 0)),
            pl.BlockSpec((_D_EMB, _HID), lambda i: (0, 0)),
            pl.BlockSpec((_HID, _D), lambda i: (0, 0)),
        ],
        out_specs=pl.BlockSpec((be, _D), lambda i: (i, 0)),
        out_shape=jax.ShapeDtypeStruct((_E, _D), jnp.float32),
    )(emb, ea, wm0, wm1)


def _node_body(x_ref, attrs_ref, wlin1_ref, wsct_ref, xl_ref, sc_ref):
    x = x_ref[...]
    a = attrs_ref[...]
    xl_ref[...] = jnp.dot(x, wlin1_ref[...],
                          preferred_element_type=jnp.float32) * (1.0 / math.sqrt(_D))
    acc = jnp.zeros_like(x)
    for j in range(_D_ATTR):
        acc = acc + jnp.dot(x * a[:, j:j + 1], wsct_ref[j],
                            preferred_element_type=jnp.float32)
    sc_ref[...] = acc * (1.0 / math.sqrt(_D * _D_ATTR))


def _node_side(x, attrs, wlin1, wsct):
    bn = 2000
    grid = _N // bn
    return pl.pallas_call(
        _node_body,
        grid=(grid,),
        in_specs=[
            pl.BlockSpec((bn, _D), lambda i: (i, 0)),
            pl.BlockSpec((bn, _D_ATTR), lambda i: (i, 0)),
            pl.BlockSpec((_D, _D), lambda i: (0, 0)),
            pl.BlockSpec((_D_ATTR, _D, _D), lambda i: (0, 0, 0)),
        ],
        out_specs=[
            pl.BlockSpec((bn, _D), lambda i: (i, 0)),
            pl.BlockSpec((bn, _D), lambda i: (i, 0)),
        ],
        out_shape=[
            jax.ShapeDtypeStruct((_N, _D), jnp.float32),
            jax.ShapeDtypeStruct((_N, _D), jnp.float32),
        ],
    )(x, attrs, wlin1, wsct)


def _final_body(p0_ref, p1_ref, sc_ref, wlin2_ref, out_ref):
    p = p0_ref[...] + p1_ref[...]
    out_ref[...] = jnp.dot(p, wlin2_ref[...],
                           preferred_element_type=jnp.float32) * (1.0 / math.sqrt(_D)) + sc_ref[...]


def _final(p0, p1, sc, wlin2):
    bn = 2000
    grid = _N // bn
    return pl.pallas_call(
        _final_body,
        grid=(grid,),
        in_specs=[
            pl.BlockSpec((bn, _D), lambda i: (i, 0)),
            pl.BlockSpec((bn, _D), lambda i: (i, 0)),
            pl.BlockSpec((bn, _D), lambda i: (i, 0)),
            pl.BlockSpec((_D, _D), lambda i: (0, 0)),
        ],
        out_specs=pl.BlockSpec((bn, _D), lambda i: (i, 0)),
        out_shape=jax.ShapeDtypeStruct((_N, _D), jnp.float32),
    )(p0, p1, sc, wlin2)


def _sc_body(xl_hbm, wcomb_hbm, idx2_hbm, zeros_hbm, out_hbm,
             idxb, rows, wv, acc, sem_i, sem_g, sem_w):
    c = lax.axis_index("c")
    s = lax.axis_index("s")
    wid = c * _NS + s

    pltpu.sync_copy(zeros_hbm.at[pl.ds(s * _STRIPE, _STRIPE)],
                    acc.at[pl.ds(s * _STRIPE, _STRIPE)])
    plsc.subcore_barrier()

    def chunk(t):
        return wid + t * _NW

    def wbase(t):
        return jnp.minimum(chunk(t) * _B, _E - _B)

    pltpu.sync_copy(idx2_hbm.at[chunk(0)], idxb.at[pl.ds(0, 2)])
    pltpu.async_copy(xl_hbm.at[idxb.at[0]], rows.at[0], sem_g)
    pltpu.async_copy(wcomb_hbm.at[pl.ds(wbase(0), _B)], wv, sem_w)

    def super_body(q, carry):
        for k in range(4):
            t = q * 4 + k
            sl = k % 2
            nsl = 1 - sl

            @pl.when(t + 1 < _NT)
            def _(t=t, sl=sl, nsl=nsl):
                pltpu.sync_copy(idx2_hbm.at[chunk(t + 1)],
                                idxb.at[pl.ds(2 * nsl, 2)])
                pltpu.async_copy(xl_hbm.at[idxb.at[2 * nsl]], rows.at[nsl],
                                 sem_g)

            pltpu.make_async_copy(xl_hbm.at[idxb.at[2 * sl]], rows.at[sl],
                                  sem_g).wait()
            pltpu.make_async_copy(wcomb_hbm.at[pl.ds(wbase(t), _B)], wv,
                                  sem_w).wait()

            rowsb = rows.at[sl]

            def _mul(i, carry2):
                for j in range(_D // 16):
                    rowsb[i, pl.ds(j * 16, 16)] = (rowsb[i, pl.ds(j * 16, 16)]
                                                   * wv[i, pl.ds(j * 16, 16)])
                return carry2

            lax.fori_loop(0, _B, _mul, 0)

            pltpu.sync_copy(rows.at[sl], acc.at[idxb.at[2 * sl + 1]],
                            add=True)

            @pl.when(t + 1 < _NT)
            def _(t=t):
                pltpu.async_copy(wcomb_hbm.at[pl.ds(wbase(t + 1), _B)], wv,
                                 sem_w)
        return carry

    lax.fori_loop(0, _NT // 4, super_body, 0)
    plsc.subcore_barrier()

    pltpu.sync_copy(acc.at[pl.ds(s * _STRIPE, _STRIPE)],
                    out_hbm.at[pl.ds(c * _NPAD + s * _STRIPE, _STRIPE)])


def _sc_scatter(xl, wcomb, idx2, zeros):
    mesh = plsc.VectorSubcoreMesh(core_axis_name="c", subcore_axis_name="s")
    f = functools.partial(
        pl.kernel,
        mesh=mesh,
        out_type=jax.ShapeDtypeStruct((_NC * _NPAD, _D), jnp.float32),
        scratch_types=[
            pltpu.VMEM((4, _B), jnp.int32),
            pltpu.VMEM((2, _B, _D), jnp.float32),
            pltpu.VMEM((_B, _D), jnp.float32),
            pltpu.VMEM_SHARED((_NPAD, _D), jnp.float32),
            pltpu.SemaphoreType.DMA,
            pltpu.SemaphoreType.DMA,
            pltpu.SemaphoreType.DMA,
        ],
    )(_sc_body)
    return f(xl, wcomb, idx2, zeros)


def kernel(node_features, node_attrs, edge_index, edge_attrs, edge_embedding,
           W_lin1, W_mlp0, W_mlp1, W_lin2, W_sc):
    npad = _EPAD - _E
    pad_ar = jnp.arange(npad, dtype=jnp.int32)
    edge_src = jnp.concatenate([edge_index[1], pad_ar % _N])
    edge_dst = jnp.concatenate([edge_index[0], _N + pad_ar % (_NPAD - _N)])
    idx2 = jnp.stack([edge_src.reshape(_NCHUNK, _B),
                      edge_dst.reshape(_NCHUNK, _B)], axis=1)
    wsct = jnp.transpose(W_sc, (1, 0, 2))
    zeros = jnp.zeros((_NPAD, _D), jnp.float32)

    wcomb = _edge_weights(edge_embedding, edge_attrs, W_mlp0, W_mlp1)
    xl, sc = _node_side(node_features, node_attrs, W_lin1, wsct)
    parts = _sc_scatter(xl, wcomb, idx2, zeros)
    return _final(parts[:_N], parts[_NPAD:_NPAD + _N], sc, W_lin2)
